# baseline (speedup 1.0000x reference)
import functools

import jax
import jax.numpy as jnp
from jax import lax
from jax.experimental import pallas as pl
from jax.experimental.pallas import tpu as pltpu
from jax.experimental.pallas import tpu_sc as plsc


def _sc_gather_prefix(idx, emb, B, H):
    mesh = plsc.VectorSubcoreMesh(core_axis_name="c", subcore_axis_name="s")

    @functools.partial(
        pl.kernel,
        out_type=jax.ShapeDtypeStruct((B, 1, H), emb.dtype),
        mesh=mesh,
        scratch_types=[
            pltpu.VMEM((B,), jnp.int32),
            pltpu.VMEM((B, H), jnp.float32),
            pltpu.SemaphoreType.DMA,
        ],
    )
    def _gather(idx_hbm, emb_hbm, pre_hbm, idx_v, rows_v, sem):
        wid = lax.axis_index("s") * 2 + lax.axis_index("c")

        @pl.when(wid == 0)
        def _():
            pltpu.sync_copy(idx_hbm, idx_v)
            pltpu.async_copy(emb_hbm.at[idx_v], rows_v, sem).wait()
            pltpu.sync_copy(rows_v, pre_hbm.at[:, 0, :])

    return _gather(idx, emb)


def _tc_assemble_body(prefix_ref, x_ref, o_ref):
    o_ref[0, 0, 0, :] = prefix_ref[0, 0, :]
    o_ref[0, 0, 1:, :] = x_ref[0, 0]


def kernel(x, resolution_idx, resolution_embeddings):
    B, C, P, H = x.shape
    idx = resolution_idx.astype(jnp.int32)

    prefix = _sc_gather_prefix(idx, resolution_embeddings, B, H)

    out = pl.pallas_call(
        _tc_assemble_body,
        grid=(B, C),
        in_specs=[
            pl.BlockSpec((1, 1, H), lambda b, c: (b, 0, 0)),
            pl.BlockSpec((1, 1, P, H), lambda b, c: (b, c, 0, 0)),
        ],
        out_specs=pl.BlockSpec((1, 1, P + 1, H), lambda b, c: (b, c, 0, 0)),
        out_shape=jax.ShapeDtypeStruct((B, C, P + 1, H), x.dtype),
    )(prefix, x)
    return out


# speedup vs baseline: 1.5627x; 1.5627x over previous
import functools

import jax
import jax.numpy as jnp
from jax import lax
from jax.experimental import pallas as pl
from jax.experimental.pallas import tpu as pltpu
from jax.experimental.pallas import tpu_sc as plsc


def _sc_gather_prefix(idx, emb, B, H):
    mesh = plsc.VectorSubcoreMesh(core_axis_name="c", subcore_axis_name="s")

    @functools.partial(
        pl.kernel,
        out_type=jax.ShapeDtypeStruct((B, 1, H), emb.dtype),
        mesh=mesh,
        scratch_types=[
            pltpu.VMEM((B,), jnp.int32),
            pltpu.VMEM((B, H), jnp.float32),
            pltpu.SemaphoreType.DMA,
        ],
    )
    def _gather(idx_hbm, emb_hbm, pre_hbm, idx_v, rows_v, sem):
        wid = lax.axis_index("s") * 2 + lax.axis_index("c")

        @pl.when(wid == 0)
        def _():
            pltpu.sync_copy(idx_hbm, idx_v)
            pltpu.async_copy(emb_hbm.at[idx_v], rows_v, sem).wait()
            pltpu.sync_copy(rows_v, pre_hbm.at[:, 0, :])

    return _gather(idx, emb)


_BLOCK_C = 8


def _tc_assemble_body(prefix_ref, x_ref, o_ref):
    row = prefix_ref[0, 0, :]
    o_ref[0, :, 0, :] = jnp.broadcast_to(row[None, :], (o_ref.shape[1], row.shape[0]))
    o_ref[0, :, 1:, :] = x_ref[0]


def kernel(x, resolution_idx, resolution_embeddings):
    B, C, P, H = x.shape
    idx = resolution_idx.astype(jnp.int32)

    prefix = _sc_gather_prefix(idx, resolution_embeddings, B, H)

    out = pl.pallas_call(
        _tc_assemble_body,
        grid=(B, C // _BLOCK_C),
        in_specs=[
            pl.BlockSpec((1, 1, H), lambda b, c: (b, 0, 0)),
            pl.BlockSpec((1, _BLOCK_C, P, H), lambda b, c: (b, c, 0, 0)),
        ],
        out_specs=pl.BlockSpec((1, _BLOCK_C, P + 1, H), lambda b, c: (b, c, 0, 0)),
        out_shape=jax.ShapeDtypeStruct((B, C, P + 1, H), x.dtype),
    )(prefix, x)
    return out


# speedup vs baseline: 1.5696x; 1.0044x over previous
import functools

import jax
import jax.numpy as jnp
from jax import lax
from jax.experimental import pallas as pl
from jax.experimental.pallas import tpu as pltpu
from jax.experimental.pallas import tpu_sc as plsc


def _sc_gather_prefix(idx, emb, B, H):
    mesh = plsc.VectorSubcoreMesh(core_axis_name="c", subcore_axis_name="s")

    @functools.partial(
        pl.kernel,
        out_type=jax.ShapeDtypeStruct((B, 1, H), emb.dtype),
        mesh=mesh,
        scratch_types=[
            pltpu.VMEM((B,), jnp.int32),
            pltpu.VMEM((B, H), jnp.float32),
            pltpu.SemaphoreType.DMA,
        ],
    )
    def _gather(idx_hbm, emb_hbm, pre_hbm, idx_v, rows_v, sem):
        wid = lax.axis_index("s") * 2 + lax.axis_index("c")

        @pl.when(wid == 0)
        def _():
            pltpu.sync_copy(idx_hbm, idx_v)
            pltpu.async_copy(emb_hbm.at[idx_v], rows_v, sem).wait()
            pltpu.sync_copy(rows_v, pre_hbm.at[:, 0, :])

    return _gather(idx, emb)


_BLOCK_C = 16


def _tc_assemble_body(prefix_ref, x_ref, o_ref):
    row = prefix_ref[0, 0, :]
    o_ref[0, :, 0, :] = jnp.broadcast_to(row[None, :], (o_ref.shape[1], row.shape[0]))
    o_ref[0, :, 1:, :] = x_ref[0]


def kernel(x, resolution_idx, resolution_embeddings):
    B, C, P, H = x.shape
    idx = resolution_idx.astype(jnp.int32)

    prefix = _sc_gather_prefix(idx, resolution_embeddings, B, H)

    out = pl.pallas_call(
        _tc_assemble_body,
        grid=(B, C // _BLOCK_C),
        in_specs=[
            pl.BlockSpec((1, 1, H), lambda b, c: (b, 0, 0)),
            pl.BlockSpec((1, _BLOCK_C, P, H), lambda b, c: (b, c, 0, 0)),
        ],
        out_specs=pl.BlockSpec((1, _BLOCK_C, P + 1, H), lambda b, c: (b, c, 0, 0)),
        out_shape=jax.ShapeDtypeStruct((B, C, P + 1, H), x.dtype),
    )(prefix, x)
    return out
